# baseline (speedup 1.0000x reference)
import functools

import jax
import jax.numpy as jnp
from jax import lax
from jax.experimental import pallas as pl
from jax.experimental.pallas import tpu as pltpu
from jax.experimental.pallas import tpu_sc as plsc

EMBED_DIM = 32
B_TOTAL = 16384 * 50
NUM_CORES = 2
NUM_SUBCORES = 16
NW = NUM_CORES * NUM_SUBCORES
B_PER_W = B_TOTAL // NW
CHUNK = 128
NCHUNK = B_PER_W // CHUNK

_mesh = plsc.VectorSubcoreMesh(core_axis_name="c", subcore_axis_name="s")


@functools.partial(
    pl.kernel,
    mesh=_mesh,
    out_type=jax.ShapeDtypeStruct((B_TOTAL, EMBED_DIM), jnp.float32),
    scratch_types=[
        pltpu.VMEM((CHUNK,), jnp.int32),
        pltpu.VMEM((CHUNK, EMBED_DIM), jnp.float32),
        pltpu.SemaphoreType.DMA,
    ],
    compiler_params=pltpu.CompilerParams(use_tc_tiling_on_sc=False),
)
def _sc_gather(idx_hbm, table_hbm, out_hbm, idx_v, rows_v, sem):
    wid = lax.axis_index("s") * NUM_CORES + lax.axis_index("c")
    base = wid * B_PER_W

    def body(i, carry):
        off = base + i * CHUNK
        pltpu.sync_copy(idx_hbm.at[pl.ds(off, CHUNK)], idx_v)
        pltpu.async_copy(table_hbm.at[idx_v], rows_v, sem).wait()
        pltpu.sync_copy(rows_v, out_hbm.at[pl.ds(off, CHUNK)])
        return carry

    lax.fori_loop(0, NCHUNK, body, 0)


def kernel(token_ids, weight):
    flat = token_ids.reshape(-1).astype(jnp.int32)
    out = _sc_gather(flat, weight)
    return out.reshape(token_ids.shape + (weight.shape[1],))


# speedup vs baseline: 1.1342x; 1.1342x over previous
import functools

import jax
import jax.numpy as jnp
from jax import lax
from jax.experimental import pallas as pl
from jax.experimental.pallas import tpu as pltpu
from jax.experimental.pallas import tpu_sc as plsc

EMBED_DIM = 32
B_TOTAL = 16384 * 50
NUM_CORES = 2
NUM_SUBCORES = 16
NW = NUM_CORES * NUM_SUBCORES
B_PER_W = B_TOTAL // NW
CHUNK = 512
NCHUNK = B_PER_W // CHUNK
NPAIR = NCHUNK // 2

_mesh = plsc.VectorSubcoreMesh(core_axis_name="c", subcore_axis_name="s")


@functools.partial(
    pl.kernel,
    mesh=_mesh,
    out_type=jax.ShapeDtypeStruct((B_TOTAL, EMBED_DIM), jnp.float32),
    scratch_types=[
        pltpu.VMEM((CHUNK,), jnp.int32),
        pltpu.VMEM((CHUNK,), jnp.int32),
        pltpu.VMEM((CHUNK, EMBED_DIM), jnp.float32),
        pltpu.VMEM((CHUNK, EMBED_DIM), jnp.float32),
        pltpu.SemaphoreType.DMA,
        pltpu.SemaphoreType.DMA,
        pltpu.SemaphoreType.DMA,
        pltpu.SemaphoreType.DMA,
        pltpu.SemaphoreType.DMA,
        pltpu.SemaphoreType.DMA,
    ],
    compiler_params=pltpu.CompilerParams(use_tc_tiling_on_sc=False),
)
def _sc_gather(idx_hbm, table_hbm, out_hbm, idx0, idx1, rows0, rows1,
               si0, si1, sg0, sg1, so0, so1):
    wid = lax.axis_index("s") * NUM_CORES + lax.axis_index("c")
    base = wid * B_PER_W
    idx_v = [idx0, idx1]
    rows_v = [rows0, rows1]
    sem_i = [si0, si1]
    sem_g = [sg0, sg1]
    sem_o = [so0, so1]

    def idx_load(g, b):
        pltpu.async_copy(idx_hbm.at[pl.ds(base + g * CHUNK, CHUNK)],
                         idx_v[b], sem_i[b])

    def gather(b):
        pltpu.async_copy(table_hbm.at[idx_v[b]], rows_v[b], sem_g[b])

    def writeback(g, b):
        pltpu.async_copy(rows_v[b], out_hbm.at[pl.ds(base + g * CHUNK, CHUNK)],
                         sem_o[b])

    def wait_i(b):
        pltpu.make_async_copy(idx_hbm.at[pl.ds(base, CHUNK)],
                              idx_v[b], sem_i[b]).wait()

    def wait_g(b):
        pltpu.make_async_copy(table_hbm.at[idx_v[b]], rows_v[b],
                              sem_g[b]).wait()

    def wait_o(b):
        pltpu.make_async_copy(rows_v[b], out_hbm.at[pl.ds(base, CHUNK)],
                              sem_o[b]).wait()

    for b in range(2):
        idx_load(b, b)
    for b in range(2):
        wait_i(b)
        gather(b)

    def pair_body(p, carry):
        g0 = p * 2
        for b in range(2):
            wait_g(b)
            writeback(g0 + b, b)
            idx_load(g0 + 2 + b, b)
        for b in range(2):
            wait_i(b)
            wait_o(b)
            gather(b)
        return carry

    lax.fori_loop(0, NPAIR - 1, pair_body, 0)

    g0 = (NPAIR - 1) * 2
    for b in range(2):
        wait_g(b)
        writeback(g0 + b, b)
    for b in range(2):
        wait_o(b)


def kernel(token_ids, weight):
    flat = token_ids.reshape(-1).astype(jnp.int32)
    out = _sc_gather(flat, weight)
    return out.reshape(token_ids.shape + (weight.shape[1],))


# speedup vs baseline: 4.1417x; 3.6517x over previous
import functools

import jax
import jax.numpy as jnp
from jax import lax
from jax.experimental import pallas as pl
from jax.experimental.pallas import tpu as pltpu
from jax.experimental.pallas import tpu_sc as plsc

NUM_EMB = 1000000
DIM = 32
BATCH = 16384
SEQ = 50
NUM_CORES = 2
NUM_SUBCORES = 16
NW = NUM_CORES * NUM_SUBCORES

BLK1_PER_W = 246
LAST_FULL = (NUM_EMB // 128) * 128 - 128
ITEMS_PER_W = 200

_mesh = plsc.VectorSubcoreMesh(core_axis_name="c", subcore_axis_name="s")
_params = pltpu.CompilerParams(use_tc_tiling_on_sc=True)

_DNUMS = lax.GatherDimensionNumbers(
    offset_dims=(), collapsed_slice_dims=(0,), start_index_map=(0,))


def _wid():
    return lax.axis_index("s") * NUM_CORES + lax.axis_index("c")


def _permute(v, idx):
    return lax.gather(v, idx[:, None], _DNUMS, (1,),
                      mode=lax.GatherScatterMode.PROMISE_IN_BOUNDS)


def _tr16(m):
    iota = lax.iota(jnp.int32, 16)
    m = list(m)
    for k in (8, 4, 2, 1):
        mask = (iota & k) == 0
        idx_m = (iota - k) & 15
        idx_p = (iota + k) & 15
        for i in range(16):
            if i & k:
                continue
            j = i | k
            a, b = m[i], m[j]
            m[i] = jnp.where(mask, a, _permute(b, idx_m))
            m[j] = jnp.where(mask, _permute(a, idx_p), b)
    return m


@functools.partial(
    pl.kernel,
    mesh=_mesh,
    out_type=jax.ShapeDtypeStruct((NUM_EMB // 4, 128), jnp.float32),
    scratch_types=[
        pltpu.VMEM((32, 128), jnp.float32),
        pltpu.VMEM((32, 128), jnp.float32),
        pltpu.VMEM((32, 128), jnp.float32),
        pltpu.VMEM((32, 128), jnp.float32),
        pltpu.SemaphoreType.DMA,
        pltpu.SemaphoreType.DMA,
        pltpu.SemaphoreType.DMA,
        pltpu.SemaphoreType.DMA,
    ],
    compiler_params=_params,
)
def _sc_linearize(wt_hbm, wtail_hbm, scr_hbm, src0, src1, dst0, dst1,
                  ss0, ss1, sd0, sd1):
    wid = _wid()
    src = [src0, src1]
    dst = [dst0, dst1]
    sem_s = [ss0, ss1]
    sem_d = [sd0, sd1]

    def off_of(i):
        raw = jnp.minimum((wid + NW * i) * 128, LAST_FULL)
        return pl.multiple_of(raw, 128)

    def load_src(i, s):
        pltpu.async_copy(wt_hbm.at[:, pl.ds(off_of(i), 128)], src[s], sem_s[s])

    def wait_src(s):
        pltpu.make_async_copy(wt_hbm.at[:, pl.ds(0, 128)], src[s],
                              sem_s[s]).wait()

    def store_dst(i, s):
        row = pl.multiple_of(off_of(i) >> 2, 32)
        pltpu.async_copy(dst[s], scr_hbm.at[pl.ds(row, 32)], sem_d[s])

    def wait_dst(s):
        pltpu.make_async_copy(dst[s], scr_hbm.at[pl.ds(0, 32)],
                              sem_d[s]).wait()

    def transpose(s):
        def b_body(b, carry):
            for a in range(2):
                sub = [src[s][16 * a + l, pl.ds(16 * b, 16)]
                       for l in range(16)]
                t16 = _tr16(sub)
                for t in range(16):
                    slot = 2 * (t % 4) + a
                    dst[s][4 * b + t // 4, pl.ds(16 * slot, 16)] = t16[t]
            return carry
        lax.fori_loop(0, 8, b_body, 0)

    for s in range(2):
        load_src(s, s)
        store_dst(s, s)

    def pair_body(p, carry):
        for s in range(2):
            i = 2 * p + s
            wait_src(s)
            wait_dst(s)
            transpose(s)
            store_dst(i, s)
            load_src(i + 2, s)
        return carry

    lax.fori_loop(0, BLK1_PER_W // 2, pair_body, 0)

    for s in range(2):
        wait_src(s)
        wait_dst(s)

    @pl.when(wid == 0)
    def _tail():
        pltpu.async_copy(wtail_hbm, src0, ss0)
        pltpu.make_async_copy(wtail_hbm, src0, ss0).wait()
        transpose(0)
        pltpu.async_copy(dst0, scr_hbm.at[pl.ds((NUM_EMB - 128) // 4, 32)],
                         sd0)
        pltpu.make_async_copy(dst0, scr_hbm.at[pl.ds(0, 32)], sd0).wait()


@functools.partial(
    pl.kernel,
    mesh=_mesh,
    out_type=jax.ShapeDtypeStruct((SEQ, DIM, BATCH), jnp.float32),
    scratch_types=[
        pltpu.VMEM((224, 128), jnp.int32),
        pltpu.VMEM((128,), jnp.int32),
        pltpu.VMEM((128,), jnp.int32),
        pltpu.VMEM((128, 128), jnp.float32),
        pltpu.VMEM((128, 128), jnp.float32),
        pltpu.VMEM((32, 128), jnp.float32),
        pltpu.VMEM((32, 128), jnp.float32),
        pltpu.SemaphoreType.DMA,
        pltpu.SemaphoreType.DMA,
        pltpu.SemaphoreType.DMA,
        pltpu.SemaphoreType.DMA,
        pltpu.SemaphoreType.DMA,
    ],
    compiler_params=_params,
)
def _sc_embed(tt_hbm, ttail_hbm, scr_hbm, out_hbm, tids, kv0, kv1,
              rows0, rows1, ob0, ob1, sti, sg0, sg1, so0, so1):
    wid = _wid()
    kv = [kv0, kv1]
    rows = [rows0, rows1]
    ob = [ob0, ob1]
    sem_g = [sg0, sg1]
    sem_o = [so0, so1]

    def col0(bb):
        return pl.multiple_of(wid * 512 + bb * 128, 128)

    for st in range(6):
        for bb in range(4):
            pltpu.async_copy(
                tt_hbm.at[pl.ds(8 * st, 8), pl.ds(col0(bb), 128)],
                tids.at[pl.ds((st * 4 + bb) * 8, 8), :], sti)
    for bb in range(4):
        pltpu.async_copy(
            ttail_hbm.at[:, pl.ds(col0(bb), 128)],
            tids.at[pl.ds(192 + bb * 8, 8), :], sti)
    for _ in range(28):
        pltpu.make_async_copy(
            tt_hbm.at[pl.ds(0, 8), pl.ds(0, 128)],
            tids.at[pl.ds(0, 8), :], sti).wait()

    def item_sb(g):
        full = g < 192
        st = jnp.where(full, g >> 5, 6)
        bb = jnp.where(full, (g >> 3) & 3, (g - 192) >> 1)
        si = jnp.where(full, g & 7, (g - 192) & 1)
        return 8 * st + si, col0(bb)

    def tid_row(g):
        return jnp.where(g < 192, g,
                         192 + ((g - 192) >> 1) * 8 + 6 + ((g - 192) & 1))

    def compute_kv(g, s):
        tr = tid_row(g)
        for m in range(8):
            kv[s][pl.ds(16 * m, 16)] = tids[tr, pl.ds(16 * m, 16)] >> 2

    def gather(s):
        pltpu.async_copy(scr_hbm.at[kv[s]], rows[s], sem_g[s])

    def wait_gather(s):
        pltpu.make_async_copy(scr_hbm.at[kv[s]], rows[s], sem_g[s]).wait()

    def extract(g, s):
        tr = tid_row(g)

        def jb_body(jb, carry):
            tokv = tids[tr, pl.ds(16 * jb, 16)]
            remv = (tokv & 3) << 5
            for h in range(2):
                sub = [rows[s][16 * jb + t, pl.ds(remv[t] + 16 * h, 16)]
                       for t in range(16)]
                t16 = _tr16(sub)
                for u in range(16):
                    ob[s][16 * h + u, pl.ds(16 * jb, 16)] = t16[u]
            return carry
        lax.fori_loop(0, 8, jb_body, 0)

    def store_out(g, s):
        sq, b0 = item_sb(g)
        pltpu.async_copy(ob[s], out_hbm.at[sq, :, pl.ds(b0, 128)], sem_o[s])

    def wait_out(s):
        pltpu.make_async_copy(ob[s], out_hbm.at[0, :, pl.ds(0, 128)],
                              sem_o[s]).wait()

    for s in range(2):
        compute_kv(s, s)
        gather(s)
        store_out(s, s)

    def pair_body(p, carry):
        for s in range(2):
            g = 2 * p + s
            wait_gather(s)
            wait_out(s)
            extract(g, s)
            store_out(g, s)
            compute_kv(jnp.minimum(g + 2, ITEMS_PER_W - 1), s)
            gather(s)
        return carry

    lax.fori_loop(0, ITEMS_PER_W // 2, pair_body, 0)

    for s in range(2):
        wait_gather(s)
        wait_out(s)


def kernel(token_ids, weight):
    wt = weight.T
    tt = token_ids.T.astype(jnp.int32)
    wtail = lax.slice(weight, (NUM_EMB - 128, 0), (NUM_EMB, DIM)).T
    ttail = lax.slice(tt, (SEQ - 8, 0), (SEQ, BATCH))
    scr = _sc_linearize(wt, wtail)
    out3 = _sc_embed(tt, ttail, scr)
    return jnp.transpose(out3, (2, 0, 1))
